# baseline (speedup 1.0000x reference)
import functools
import math

import jax
import jax.numpy as jnp
from jax import lax
from jax.experimental import pallas as pl
from jax.experimental.pallas import tpu as pltpu
from jax.experimental.pallas import tpu_sc as plsc

D_MODEL = 64
SCALE = math.sqrt(D_MODEL)
CHUNK = 128


@functools.lru_cache(maxsize=None)
def _build_lookup(n_rows: int, d: int):
    info = plsc.get_sparse_core_info()
    nc, ns = info.num_cores, info.num_subcores
    nw = nc * ns
    assert n_rows % nw == 0
    rows_per_w = n_rows // nw

    mesh = plsc.VectorSubcoreMesh(core_axis_name="c", subcore_axis_name="s")

    @functools.partial(
        pl.kernel,
        mesh=mesh,
        out_type=jax.ShapeDtypeStruct((n_rows * CHUNK, d), jnp.float32),
        scratch_types=[
            pltpu.VMEM((rows_per_w, CHUNK), jnp.int32),
            pltpu.VMEM((CHUNK, d), jnp.float32),
            pltpu.SemaphoreType.DMA,
        ],
        compiler_params=pltpu.CompilerParams(use_tc_tiling_on_sc=False),
    )
    def lookup(idx_hbm, table_hbm, out_hbm, idx_v, buf, sem):
        wid = lax.axis_index("s") * nc + lax.axis_index("c")
        rbase = wid * rows_per_w
        pltpu.sync_copy(idx_hbm.at[wid], idx_v)

        def chunk_body(g, carry):
            cp = pltpu.make_async_copy(table_hbm.at[idx_v.at[g]], buf, sem)
            cp.start()
            cp.wait()

            def row_body(j, c2):
                for k in range(d // 16):
                    sl = pl.ds(k * 16, 16)
                    buf[j, sl] = buf[j, sl] * SCALE
                return c2

            lax.fori_loop(0, CHUNK, row_body, 0)
            pltpu.sync_copy(buf, out_hbm.at[pl.ds((rbase + g) * CHUNK, CHUNK)])
            return carry

        lax.fori_loop(0, rows_per_w, chunk_body, 0)

    return lookup


def kernel(x, table):
    b, s = x.shape
    n = b * s
    d = table.shape[1]
    info = plsc.get_sparse_core_info()
    nw = info.num_cores * info.num_subcores
    idx = x.reshape(nw, n // (CHUNK * nw), CHUNK).astype(jnp.int32)
    out = _build_lookup(n // CHUNK, d)(idx, table)
    return out.reshape(b, s, d)
